# baseline (speedup 1.0000x reference)
import functools

import jax
import jax.numpy as jnp
from jax import lax
from jax.experimental import pallas as pl
from jax.experimental.pallas import tpu as pltpu
from jax.experimental.pallas import tpu_sc as plsc

D_MODEL = 1024
LANES = 16

NUM_CORES = 2
NUM_SUBCORES = 16
NW = NUM_CORES * NUM_SUBCORES

COL_GROUPS = 4
COLS = D_MODEL // COL_GROUPS
TOK_GROUPS = NW // COL_GROUPS
TB = 32


@functools.partial(jax.jit, static_argnames=("L",))
def _pos_encode(pos_x_rs, pos_y_rs, stab_rs, tx, ty, ts, *, L):
    tok_per_w = L // TOK_GROUPS
    n_blocks = tok_per_w // TB
    n_rows = pos_x_rs.shape[1]
    mesh = plsc.VectorSubcoreMesh(core_axis_name="c", subcore_axis_name="s")

    @functools.partial(
        pl.kernel,
        out_type=jax.ShapeDtypeStruct((L, D_MODEL), jnp.float32),
        mesh=mesh,
        compiler_params=pltpu.CompilerParams(needs_layout_passes=False),
        scratch_types=[
            pltpu.VMEM((tok_per_w,), jnp.int32),
            pltpu.VMEM((tok_per_w,), jnp.int32),
            pltpu.VMEM((tok_per_w,), jnp.int32),
            pltpu.VMEM((n_rows, COLS), jnp.float32),
            pltpu.VMEM((n_rows, COLS), jnp.float32),
            pltpu.VMEM((2, COLS), jnp.float32),
            [pltpu.VMEM((TB, COLS), jnp.float32)] * 2,
            [pltpu.SemaphoreType.DMA] * 2,
        ],
    )
    def body(pos_x_hbm, pos_y_hbm, stab_hbm, tx_hbm, ty_hbm, ts_hbm, out_hbm,
             tx_v, ty_v, ts_v, px_s, py_s, stab_s, stg, sem_o):
        wid = lax.axis_index("s") * NUM_CORES + lax.axis_index("c")
        tg = wid // COL_GROUPS
        cg = wid % COL_GROUPS
        tok0 = tg * tok_per_w

        pltpu.sync_copy(tx_hbm.at[pl.ds(tok0, tok_per_w)], tx_v)
        pltpu.sync_copy(ty_hbm.at[pl.ds(tok0, tok_per_w)], ty_v)
        pltpu.sync_copy(ts_hbm.at[pl.ds(tok0, tok_per_w)], ts_v)
        pltpu.sync_copy(pos_x_hbm.at[cg], px_s)
        pltpu.sync_copy(pos_y_hbm.at[cg], py_s)
        pltpu.sync_copy(stab_hbm.at[cg], stab_s)

        iota = lax.iota(jnp.int32, LANES)
        n_slices = COLS // LANES
        stab0 = [stab_s[0, pl.ds(j * LANES, LANES)] for j in range(n_slices)]
        stab1 = [stab_s[1, pl.ds(j * LANES, LANES)] for j in range(n_slices)]

        def out_window(blk):
            return out_hbm.at[pl.ds(tok0 + blk * TB, TB),
                              pl.ds(cg * COLS, COLS)]

        def pair_body(blk2, _):
            for s in range(2):
                blk = blk2 * 2 + s

                @pl.when(blk2 > 0)
                def _():
                    pltpu.make_async_copy(stg[s], out_window(blk),
                                          sem_o[s]).wait()

                def token_body(t, _):
                    tvec = jnp.full((LANES,), blk * TB + t, jnp.int32)
                    xrow = plsc.load_gather(tx_v, [tvec])
                    yrow = plsc.load_gather(ty_v, [tvec])
                    srow = plsc.load_gather(ts_v, [tvec])
                    smask = srow == 0
                    batch = 16
                    for j0 in range(0, n_slices, batch):
                        vals = []
                        for j in range(j0, j0 + batch):
                            colj = j * LANES + iota
                            a = plsc.load_gather(px_s, [xrow, colj])
                            b = plsc.load_gather(py_s, [yrow, colj])
                            sv = jnp.where(smask, stab0[j], stab1[j])
                            vals.append(a + b + sv)
                        for j, v in zip(range(j0, j0 + batch), vals):
                            stg[s][t, pl.ds(j * LANES, LANES)] = v
                    return 0

                lax.fori_loop(0, TB, token_body, 0)
                pltpu.async_copy(stg[s], out_window(blk), sem_o[s])
            return 0

        lax.fori_loop(0, n_blocks // 2, pair_body, 0)

        for s in range(2):
            pltpu.make_async_copy(stg[s], out_window(s), sem_o[s]).wait()

    return body(pos_x_rs, pos_y_rs, stab_rs, tx, ty, ts)


def kernel(x, pos_x, pos_y, stab, token_to_x, token_to_y, token_to_stab):
    L = x.shape[1]
    tx = token_to_x[:L].astype(jnp.int32)
    ty = token_to_y[:L].astype(jnp.int32)
    ts = token_to_stab[:L].astype(jnp.int32)
    pos_x_rs = pos_x.reshape(-1, COL_GROUPS, COLS).transpose(1, 0, 2)
    pos_y_rs = pos_y.reshape(-1, COL_GROUPS, COLS).transpose(1, 0, 2)
    stab_rs = stab.reshape(-1, COL_GROUPS, COLS).transpose(1, 0, 2)
    return _pos_encode(pos_x_rs, pos_y_rs, stab_rs, tx, ty, ts, L=L)


# speedup vs baseline: 1.0983x; 1.0983x over previous
import functools

import jax
import jax.numpy as jnp
from jax import lax
from jax.experimental import pallas as pl
from jax.experimental.pallas import tpu as pltpu
from jax.experimental.pallas import tpu_sc as plsc

D_MODEL = 1024
LANES = 16

NUM_CORES = 2
NUM_SUBCORES = 16
NW = NUM_CORES * NUM_SUBCORES

COL_GROUPS = 4
COLS = D_MODEL // COL_GROUPS
TOK_GROUPS = NW // COL_GROUPS
TB = 32


@functools.partial(jax.jit, static_argnames=("L",))
def _pos_encode(pos_x_rs, pos_y_rs, stab_rs, tx, ty, ts, *, L):
    tok_per_w = L // TOK_GROUPS
    n_blocks = tok_per_w // TB
    n_rows = pos_x_rs.shape[1]
    mesh = plsc.VectorSubcoreMesh(core_axis_name="c", subcore_axis_name="s")

    @functools.partial(
        pl.kernel,
        out_type=jax.ShapeDtypeStruct((L, D_MODEL), jnp.float32),
        mesh=mesh,
        compiler_params=pltpu.CompilerParams(needs_layout_passes=False),
        scratch_types=[
            pltpu.VMEM((tok_per_w,), jnp.int32),
            pltpu.VMEM((tok_per_w,), jnp.int32),
            pltpu.VMEM((tok_per_w,), jnp.int32),
            pltpu.VMEM((n_rows, COLS), jnp.float32),
            pltpu.VMEM((n_rows, COLS), jnp.float32),
            pltpu.VMEM((2, COLS), jnp.float32),
            [pltpu.VMEM((TB, COLS), jnp.float32)] * 2,
            [pltpu.SemaphoreType.DMA] * 2,
        ],
    )
    def body(pos_x_hbm, pos_y_hbm, stab_hbm, tx_hbm, ty_hbm, ts_hbm, out_hbm,
             tx_v, ty_v, ts_v, px_s, py_s, stab_s, stg, sem_o):
        wid = lax.axis_index("s") * NUM_CORES + lax.axis_index("c")
        tg = wid // COL_GROUPS
        cg = wid % COL_GROUPS
        tok0 = tg * tok_per_w

        pltpu.sync_copy(tx_hbm.at[pl.ds(tok0, tok_per_w)], tx_v)
        pltpu.sync_copy(ty_hbm.at[pl.ds(tok0, tok_per_w)], ty_v)
        pltpu.sync_copy(ts_hbm.at[pl.ds(tok0, tok_per_w)], ts_v)
        pltpu.sync_copy(pos_x_hbm.at[cg], px_s)
        pltpu.sync_copy(pos_y_hbm.at[cg], py_s)
        pltpu.sync_copy(stab_hbm.at[cg], stab_s)

        iota = lax.iota(jnp.int32, LANES)
        n_slices = COLS // LANES
        stab0 = [stab_s[0, pl.ds(j * LANES, LANES)] for j in range(n_slices)]
        stab1 = [stab_s[1, pl.ds(j * LANES, LANES)] for j in range(n_slices)]

        def out_window(blk):
            return out_hbm.at[pl.ds(tok0 + blk * TB, TB),
                              pl.ds(cg * COLS, COLS)]

        def pair_body(blk2, _):
            for s in range(2):
                blk = blk2 * 2 + s

                @pl.when(blk2 > 0)
                def _():
                    pltpu.make_async_copy(stg[s], out_window(blk),
                                          sem_o[s]).wait()

                def group_body(g, _):
                    gb = blk * TB + g * LANES
                    xr = tx_v[pl.ds(gb, LANES)]
                    yr = ty_v[pl.ds(gb, LANES)]
                    sr = ts_v[pl.ds(gb, LANES)]

                    def token_body(tt, _):
                        tsplat = jnp.full((LANES,), tt, jnp.int32)
                        xrow = xr.at[tsplat].get(mode="promise_in_bounds")
                        yrow = yr.at[tsplat].get(mode="promise_in_bounds")
                        srow = sr.at[tsplat].get(mode="promise_in_bounds")
                        smask = srow == 0
                        t = g * LANES + tt
                        batch = 8
                        for j0 in range(0, n_slices, batch):
                            vals = []
                            for j in range(j0, j0 + batch):
                                colj = j * LANES + iota
                                a = plsc.load_gather(px_s, [xrow, colj])
                                b = plsc.load_gather(py_s, [yrow, colj])
                                sv = jnp.where(smask, stab0[j], stab1[j])
                                vals.append(a + b + sv)
                            for j, v in zip(range(j0, j0 + batch), vals):
                                stg[s][t, pl.ds(j * LANES, LANES)] = v
                        return 0

                    lax.fori_loop(0, LANES, token_body, 0)
                    return 0

                lax.fori_loop(0, TB // LANES, group_body, 0)
                pltpu.async_copy(stg[s], out_window(blk), sem_o[s])
            return 0

        lax.fori_loop(0, n_blocks // 2, pair_body, 0)

        for s in range(2):
            pltpu.make_async_copy(stg[s], out_window(s), sem_o[s]).wait()

    return body(pos_x_rs, pos_y_rs, stab_rs, tx, ty, ts)


def kernel(x, pos_x, pos_y, stab, token_to_x, token_to_y, token_to_stab):
    L = x.shape[1]
    tx = token_to_x[:L].astype(jnp.int32)
    ty = token_to_y[:L].astype(jnp.int32)
    ts = token_to_stab[:L].astype(jnp.int32)
    pos_x_rs = pos_x.reshape(-1, COL_GROUPS, COLS).transpose(1, 0, 2)
    pos_y_rs = pos_y.reshape(-1, COL_GROUPS, COLS).transpose(1, 0, 2)
    stab_rs = stab.reshape(-1, COL_GROUPS, COLS).transpose(1, 0, 2)
    return _pos_encode(pos_x_rs, pos_y_rs, stab_rs, tx, ty, ts, L=L)
